# Initial kernel scaffold; baseline (speedup 1.0000x reference)
import jax
import jax.numpy as jnp
from jax.experimental import pallas as pl


def kernel(feature_idx, hist_item_idx, hist_author_idx, hist_music_idx, W_emb, W1, b1, W2, b2, W3, b3, Wo, bo):
    raise NotImplementedError("write your pallas kernel here")


# speedup vs baseline: 3.4699x; 3.4699x over previous
import functools

import jax
import jax.numpy as jnp
from jax import lax
from jax.experimental import pallas as pl
from jax.experimental.pallas import tpu as pltpu
from jax.experimental.pallas import tpu_sc as plsc

FEATURE_PAD = 100000
EMB = 64
B = 4096
NW = 32
ROWS_PER_W = B // NW
MLP_BLOCK = 256

_SEQS = (
    (0, 350, 384, 3 + 2),
    (1, 250, 256, 3 + 3),
    (2, 100, 128, 3 + 6),
)

_F32 = jnp.float32
_I32 = jnp.int32


def _splat(vec, j):
    return vec.at[jnp.full((16,), j, _I32)].get(mode="promise_in_bounds")


def _din_sc_body(W_emb, fidx_h, item_h, auth_h, music_h, out,
                 idxbuf, fidxbuf, hbuf, sbuf, xrow, sem):
    wid = lax.axis_index("s") * 2 + lax.axis_index("c")
    hists = (item_h, auth_h, music_h)
    iota16 = lax.iota(_I32, 16)

    def row_body(i, carry):
        b = wid * ROWS_PER_W + i
        pltpu.sync_copy(fidx_h.at[b], fidxbuf)
        pltpu.async_copy(W_emb.at[fidxbuf], xrow.at[pl.ds(3, 8)], sem).wait()

        for seq_i, (slot, L, Lp, arow) in enumerate(_SEQS):
            idx_h = hists[seq_i]
            pltpu.sync_copy(idx_h.at[b], idxbuf.at[pl.ds(0, Lp)])
            cps = [
                pltpu.async_copy(
                    W_emb.at[idxbuf.at[pl.ds(c * 128, 128)]],
                    hbuf.at[pl.ds(c * 128, 128)], sem)
                for c in range(Lp // 128)
            ]
            for cp in cps:
                cp.wait()

            ngrp = (L + 15) // 16
            a_chunks = [xrow[arow, pl.ds(c * 16, 16)] for c in range(4)]

            def sgroup(g, carry):
                lvec = g * 16 + iota16
                acc = jnp.zeros((16,), _F32)
                for c in range(4):
                    for j in range(16):
                        k = c * 16 + j
                        col = plsc.load_gather(
                            hbuf, [lvec, jnp.full((16,), k, _I32)])
                        acc = acc + col * _splat(a_chunks[c], j)
                idxv = idxbuf[pl.ds(g * 16, 16)]
                sv = jnp.where(idxv != FEATURE_PAD, acc, _F32(-1e9))
                sbuf[pl.ds(g * 16, 16)] = sv
                return carry

            lax.fori_loop(0, ngrp, sgroup, 0, unroll=False)

            def mstep(g, mv):
                return jnp.maximum(mv, sbuf[pl.ds(g * 16, 16)])

            mv = lax.fori_loop(0, ngrp, mstep,
                               jnp.full((16,), -3e38, _F32), unroll=False)
            m = _splat(plsc.cummax(mv), 15)

            def estep(g, dv):
                sv = sbuf[pl.ds(g * 16, 16)]
                pos = g * 16 + iota16
                ev = jnp.where(pos < L, jnp.exp(sv - m), _F32(0.0))
                sbuf[pl.ds(g * 16, 16)] = ev
                return dv + ev

            dv = lax.fori_loop(0, ngrp, estep, jnp.zeros((16,), _F32),
                               unroll=False)
            inv = 1.0 / _splat(plsc.cumsum(dv), 15)

            def wstep(t, nums):
                n0, n1, n2, n3 = nums
                ev = sbuf[pl.ds(t * 16, 16)]
                for j in range(16):
                    l = t * 16 + j
                    e_l = _splat(ev, j)
                    n0 = n0 + hbuf[l, pl.ds(0, 16)] * e_l
                    n1 = n1 + hbuf[l, pl.ds(16, 16)] * e_l
                    n2 = n2 + hbuf[l, pl.ds(32, 16)] * e_l
                    n3 = n3 + hbuf[l, pl.ds(48, 16)] * e_l
                return (n0, n1, n2, n3)

            z = jnp.zeros((16,), _F32)
            nums = lax.fori_loop(0, ngrp, wstep, (z, z, z, z),
                                 unroll=False)
            for c, n in enumerate(nums):
                xrow[slot, pl.ds(c * 16, 16)] = n * inv

        pltpu.sync_copy(xrow, out.at[b])
        return carry

    lax.fori_loop(0, ROWS_PER_W, row_body, 0, unroll=False)


@functools.cache
def _din_sc():
    return pl.kernel(
        _din_sc_body,
        out_type=jax.ShapeDtypeStruct((B, 11, EMB), _F32),
        mesh=plsc.VectorSubcoreMesh(core_axis_name="c", subcore_axis_name="s",
                                    num_cores=2, num_subcores=16),
        scratch_types=[
            pltpu.VMEM((384,), _I32),
            pltpu.VMEM((8,), _I32),
            pltpu.VMEM((384, EMB), _F32),
            pltpu.VMEM((384,), _F32),
            pltpu.VMEM((11, EMB), _F32),
            pltpu.SemaphoreType.DMA,
        ],
        compiler_params=pltpu.CompilerParams(needs_layout_passes=False,
                                             use_tc_tiling_on_sc=False),
    )


def _mlp_body(x_ref, w1_ref, b1_ref, w2_ref, b2_ref, w3_ref, b3_ref,
              wo_ref, bo_ref, out_ref):
    dot = functools.partial(jax.lax.dot_general,
                            dimension_numbers=(((1,), (0,)), ((), ())),
                            preferred_element_type=_F32,
                            precision=jax.lax.Precision.HIGHEST)
    h = jnp.maximum(dot(x_ref[...], w1_ref[...]) + b1_ref[...], 0.0)
    h = jnp.maximum(dot(h, w2_ref[...]) + b2_ref[...], 0.0)
    h = jnp.maximum(dot(h, w3_ref[...]) + b3_ref[...], 0.0)
    out_ref[...] = dot(h, wo_ref[...]) + bo_ref[...]


def _mlp(x, W1, b1, W2, b2, W3, b3, Wo, bo):
    nblk = B // MLP_BLOCK
    full = lambda i: (0, 0)
    return pl.pallas_call(
        _mlp_body,
        grid=(nblk,),
        in_specs=[
            pl.BlockSpec((MLP_BLOCK, x.shape[1]), lambda i: (i, 0)),
            pl.BlockSpec(W1.shape, full),
            pl.BlockSpec((1, b1.shape[0]), full),
            pl.BlockSpec(W2.shape, full),
            pl.BlockSpec((1, b2.shape[0]), full),
            pl.BlockSpec(W3.shape, full),
            pl.BlockSpec((1, b3.shape[0]), full),
            pl.BlockSpec(Wo.shape, full),
            pl.BlockSpec((1, 1), full),
        ],
        out_specs=pl.BlockSpec((MLP_BLOCK, 1), lambda i: (i, 0)),
        out_shape=jax.ShapeDtypeStruct((B, 1), _F32),
    )(x, W1, b1.reshape(1, -1), W2, b2.reshape(1, -1),
      W3, b3.reshape(1, -1), Wo, bo.reshape(1, 1))


def kernel(feature_idx, hist_item_idx, hist_author_idx, hist_music_idx,
           W_emb, W1, b1, W2, b2, W3, b3, Wo, bo):
    item_p = jnp.pad(hist_item_idx.astype(_I32), ((0, 0), (0, 384 - 350)),
                     constant_values=FEATURE_PAD)
    auth_p = jnp.pad(hist_author_idx.astype(_I32), ((0, 0), (0, 256 - 250)),
                     constant_values=FEATURE_PAD)
    music_p = jnp.pad(hist_music_idx.astype(_I32), ((0, 0), (0, 128 - 100)),
                      constant_values=FEATURE_PAD)
    x3 = _din_sc()(W_emb, feature_idx.astype(_I32), item_p, auth_p, music_p)
    x = x3.reshape(B, 11 * EMB)
    return _mlp(x, W1, b1, W2, b2, W3, b3, Wo, bo)
